# baseline (speedup 1.0000x reference)
import functools

import jax
import jax.numpy as jnp
import numpy as np
from jax import lax
from jax.experimental import pallas as pl
from jax.experimental.pallas import tpu as pltpu
from jax.experimental.pallas import tpu_sc as plsc

_K = 64
_B = 128
_N = 32768
_L = 16
_VPB = 4
_BLK = _L * _VPB
_NBLK = _N // _BLK
_BIG = np.int32(2**30)


def _build_sc_select():
    info = plsc.get_sparse_core_info()
    NC, NS = info.num_cores, info.num_subcores
    NW = NC * NS
    RPW = _B // NW

    mesh = plsc.VectorSubcoreMesh(core_axis_name="c", subcore_axis_name="s")

    @functools.partial(
        pl.kernel,
        out_type=[
            jax.ShapeDtypeStruct((NW, _L), jnp.float32),
            jax.ShapeDtypeStruct((NW, _L), jnp.int32),
        ],
        mesh=mesh,
        compiler_params=pltpu.CompilerParams(needs_layout_passes=False),
        scratch_types=[
            pltpu.VMEM((_N,), jnp.float32),
            pltpu.VMEM((_N + _BLK,), jnp.float32),
            pltpu.VMEM((_L,), jnp.float32),
            pltpu.VMEM((_L,), jnp.int32),
        ],
    )
    def sc_select(scores_hbm, thr_hbm, tie_hbm, row_v, cand_v, sthr_v, stie_v):
        wid = lax.axis_index("s") * NC + lax.axis_index("c")

        zf = jnp.zeros((_L,), jnp.float32)
        zi = jnp.zeros((_L,), jnp.int32)
        oi = jnp.ones((_L,), jnp.int32)
        nanv = jnp.full((_L,), jnp.nan, jnp.float32)
        lane = lax.broadcasted_iota(jnp.int32, (_L,), 0)
        stage_thr = zf
        stage_tie = zi

        for r in range(RPW):
            row_idx = wid * RPW + r
            pltpu.sync_copy(scores_hbm.at[row_idx], row_v)

            def pre_body(i, carry):
                s, mn, mx = carry
                base = i * _BLK
                for j in range(_VPB):
                    v = row_v[pl.ds(base + j * _L, _L)]
                    cand_v[pl.ds(base + j * _L, _L)] = v
                    s = s + v
                    mn = jnp.minimum(mn, v)
                    mx = jnp.maximum(mx, v)
                return s, mn, mx

            sv, mnv, mxv = lax.fori_loop(
                0, _NBLK, pre_body,
                (zf, jnp.full((_L,), np.float32(np.inf), jnp.float32),
                 jnp.full((_L,), np.float32(-np.inf), jnp.float32)))
            row_sum = jnp.sum(sv)
            row_min = jnp.min(mnv)
            row_max = jnp.max(mxv)
            pivot0 = jnp.clip(row_sum * np.float32(1.0 / _N), row_min, row_max)

            def qs_cond(c):
                return jnp.logical_not(c[6])

            def qs_body(c):
                m, need, pivot, sum_cur, lo, hi, done, vk, need_eq = c
                nblk = (m + (_BLK - 1)) // _BLK

                def cnt_body(i, carry):
                    cg, cge, sg = carry
                    base = i * _BLK
                    for j in range(_VPB):
                        v = cand_v[pl.ds(base + j * _L, _L)]
                        g = v > pivot
                        ge = v >= pivot
                        cg = cg + jnp.where(g, oi, zi)
                        cge = cge + jnp.where(ge, oi, zi)
                        sg = sg + jnp.where(g, v, zf)
                    return cg, cge, sg

                cgv, cgev, sgv = lax.fori_loop(0, nblk, cnt_body, (zi, zi, zf))
                cg = jnp.sum(cgv)
                cge = jnp.sum(cgev)
                sg = jnp.sum(sgv)

                gb = cg >= need
                ex = jnp.logical_and(jnp.logical_not(gb), cge >= need)
                sum_l = sum_cur - sg - pivot * (cge - cg).astype(jnp.float32)

                gbv = jnp.broadcast_to(gb, (_L,))

                def cpt_body(i, w):
                    base = i * _BLK
                    for j in range(_VPB):
                        v = cand_v[pl.ds(base + j * _L, _L)]
                        keep = jnp.where(gbv, v > pivot, v < pivot)
                        plsc.store_compressed(cand_v.at[pl.ds(w, _L)], v,
                                              mask=keep)
                        pc = plsc.all_reduce_population_count(keep)
                        w = w + jnp.max(pc)
                    return w

                w_end = lax.fori_loop(0, nblk, cpt_body, jnp.int32(0))
                for j in range(_VPB):
                    cand_v[pl.ds(w_end + j * _L, _L)] = nanv

                m2 = jnp.where(gb, cg, m - cge)
                need2 = jnp.where(gb, need, need - cge)
                sum2 = jnp.where(gb, sg, sum_l)
                lo2 = jnp.where(gb, pivot, lo)
                hi2 = jnp.where(gb, hi, pivot)
                noprog = jnp.logical_or(
                    jnp.logical_and(gb, cg == m),
                    jnp.logical_and(jnp.logical_not(gb), cge == 0))
                mean2 = jnp.max(
                    jnp.broadcast_to(sum2, (_L,))
                    / jnp.broadcast_to(jnp.maximum(m2, 1).astype(jnp.float32),
                                       (_L,)))
                piv2 = jnp.clip(mean2, lo2, hi2)
                piv2 = jnp.where(noprog,
                                 lo2 * np.float32(0.5) + hi2 * np.float32(0.5),
                                 piv2)
                done2 = jnp.logical_or(done, ex)
                vk2 = jnp.where(ex, pivot, vk)
                need_eq2 = jnp.where(ex, need - cg, need_eq)
                return (m2, need2, piv2, sum2, lo2, hi2, done2, vk2, need_eq2)

            init = (jnp.int32(_N), jnp.int32(_K), pivot0, row_sum,
                    row_min, row_max, jnp.bool_(False),
                    jnp.float32(0.0), jnp.int32(0))
            res = lax.while_loop(qs_cond, qs_body, init)
            vk = res[7]
            need_eq = res[8]

            def tie_cond(c):
                return jnp.logical_not(c[2])

            def tie_body(c):
                blk, cnt, found = c
                base = blk * _BLK
                pc = zi
                for j in range(_VPB):
                    v = row_v[pl.ds(base + j * _L, _L)]
                    pc = pc + jnp.where(v == vk, oi, zi)
                bc = jnp.sum(pc)
                hit = cnt + bc >= need_eq
                blk2 = jnp.where(hit, blk, blk + 1)
                cnt2 = jnp.where(hit, cnt, cnt + bc)
                return blk2, cnt2, hit

            blk_f, cnt_f, _ = lax.while_loop(
                tie_cond, tie_body, (jnp.int32(0), jnp.int32(0),
                                     jnp.bool_(False)))

            base = blk_f * _BLK
            run = cnt_f
            ir = jnp.int32(_BIG)
            for j in range(_VPB):
                v = row_v[pl.ds(base + j * _L, _L)]
                e = v == vk
                ei = jnp.where(e, oi, zi)
                pr = plsc.cumsum(ei)
                tgt = jnp.logical_and(e, (run + pr) == need_eq)
                cand_ir = jnp.where(tgt, base + j * _L + lane,
                                    jnp.full((_L,), _BIG, jnp.int32))
                ir = jnp.minimum(ir, jnp.min(cand_ir))
                run = run + jnp.sum(ei)

            sel = lane == r
            stage_thr = jnp.where(sel, jnp.broadcast_to(vk, (_L,)), stage_thr)
            stage_tie = jnp.where(sel, jnp.broadcast_to(ir, (_L,)), stage_tie)

        sthr_v[pl.ds(0, _L)] = stage_thr
        stie_v[pl.ds(0, _L)] = stage_tie
        pltpu.sync_copy(sthr_v, thr_hbm.at[wid])
        pltpu.sync_copy(stie_v, tie_hbm.at[wid])

    return sc_select


_sc_select = _build_sc_select()


def _tc_apply_kernel(x_ref, thr_ref, tie_ref, masked_ref, mask_ref):
    x = x_ref[...]
    R, N = x.shape
    vk = thr_ref[...]
    tie = tie_ref[...]
    col = lax.broadcasted_iota(jnp.int32, (R, N), 1)
    mask = (x > vk) | ((x == vk) & (col <= tie))
    mask_ref[...] = mask
    masked_ref[...] = x * mask.astype(jnp.float32)


@jax.jit
def _run(scores):
    B, N = scores.shape
    thr2, tie2 = _sc_select(scores)
    rpw = B // thr2.shape[0]
    thr = thr2[:, :rpw].reshape(B, 1)
    tie = tie2[:, :rpw].reshape(B, 1)
    R = 8
    masked, mask = pl.pallas_call(
        _tc_apply_kernel,
        grid=(B // R,),
        in_specs=[
            pl.BlockSpec((R, N), lambda b: (b, 0)),
            pl.BlockSpec((R, 1), lambda b: (b, 0)),
            pl.BlockSpec((R, 1), lambda b: (b, 0)),
        ],
        out_specs=[
            pl.BlockSpec((R, N), lambda b: (b, 0)),
            pl.BlockSpec((R, N), lambda b: (b, 0)),
        ],
        out_shape=[
            jax.ShapeDtypeStruct((B, N), jnp.float32),
            jax.ShapeDtypeStruct((B, N), jnp.bool_),
        ],
    )(scores, thr, tie)
    return masked, mask


def kernel(scores, k):
    return _run(scores)
